# baseline (speedup 1.0000x reference)
import functools

import jax
import jax.numpy as jnp
from jax import lax
from jax.experimental import pallas as pl
from jax.experimental.pallas import tpu as pltpu
from jax.experimental.pallas import tpu_sc as plsc

VOCAB = 100000
EMBED = 64
BATCH = 1024

_NC = 2
_NS = 16

_N_WIN = 8
_WIN = 12544
_LAST_OFF = 87552
_N_GROUPS = BATCH // 16
_N_LAYER = _N_WIN // 2

_TILE_V = 6272
_GRID_V = (VOCAB + _TILE_V - 1) // _TILE_V


def _make_gather():
    mesh = plsc.VectorSubcoreMesh(core_axis_name="c", subcore_axis_name="s")

    @functools.partial(
        pl.kernel,
        mesh=mesh,
        out_type=jax.ShapeDtypeStruct((_N_LAYER, EMBED, BATCH), jnp.float32),
        scratch_types=[
            pltpu.VMEM((BATCH,), jnp.int32),
            pltpu.VMEM((8, _WIN), jnp.float32),
            pltpu.VMEM((1, 8, BATCH), jnp.float32),
        ],
        compiler_params=pltpu.CompilerParams(
            use_tc_tiling_on_sc=True,
            disable_bounds_checks=True,
            needs_layout_passes=False,
        ),
    )
    def gather(table_t_hbm, idx_hbm, part_hbm, idx_v, buf_v, out_v):
        cid = lax.axis_index("c")
        sid = lax.axis_index("s")
        row0 = 8 * (4 * cid + sid // 4)
        pltpu.sync_copy(idx_hbm, idx_v)

        for qi in range(2):
            q = 2 * (sid % 4) + qi
            off = jnp.where(q == _N_WIN - 1, _LAST_OFF, q * _WIN)
            own_lo = q * _WIN
            own_hi = jnp.where(
                q == _N_WIN - 2,
                _LAST_OFF,
                jnp.where(q == _N_WIN - 1, VOCAB, (q + 1) * _WIN),
            )
            own_lo = jnp.where(q == _N_WIN - 1, _LAST_OFF, own_lo)
            pltpu.sync_copy(
                table_t_hbm.at[pl.ds(row0, 8), pl.ds(off, _WIN)], buf_v
            )

            def body(g, carry, qi=qi, off=off, own_lo=own_lo, own_hi=own_hi):
                tv = idx_v[pl.ds(g * 16, 16)]
                m = (tv >= own_lo) & (tv < own_hi)
                loc = jnp.clip(tv - off, 0, _WIN - 1)
                for d in range(8):
                    dv = jnp.full((16,), d, jnp.int32)
                    v = plsc.load_gather(buf_v, [dv, loc], mask=m)
                    v = jnp.where(m, v, 0.0)
                    if qi == 0:
                        out_v[0, d, pl.ds(g * 16, 16)] = v
                    else:
                        out_v[0, d, pl.ds(g * 16, 16)] = (
                            out_v[0, d, pl.ds(g * 16, 16)] + v
                        )
                return carry

            lax.fori_loop(0, _N_GROUPS, body, 0)

            if qi == 1:
                pltpu.sync_copy(
                    out_v,
                    part_hbm.at[pl.ds(sid % 4, 1), pl.ds(row0, 8), :],
                )

    return gather


_gather = _make_gather()


def _matmul_body(w_ref, part_ref, out_ref, emb_ref):
    @pl.when(pl.program_id(0) == 0)
    def _():
        emb_ref[...] = jnp.sum(part_ref[...], axis=0)

    out_ref[...] = lax.dot_general(
        w_ref[...],
        emb_ref[...],
        (((0,), (0,)), ((), ())),
        preferred_element_type=jnp.float32,
    )


_matmul = pl.pallas_call(
    _matmul_body,
    grid=(_GRID_V,),
    in_specs=[
        pl.BlockSpec((EMBED, _TILE_V), lambda j: (0, j)),
        pl.BlockSpec((_N_LAYER, EMBED, BATCH), lambda j: (0, 0, 0)),
    ],
    out_specs=pl.BlockSpec((_TILE_V, BATCH), lambda j: (j, 0)),
    out_shape=jax.ShapeDtypeStruct((VOCAB, BATCH), jnp.float32),
    scratch_shapes=[pltpu.VMEM((EMBED, BATCH), jnp.float32)],
    compiler_params=pltpu.CompilerParams(
        dimension_semantics=("arbitrary",),
    ),
)


def kernel(token_ids, emb_table, lm_head_w):
    idx = token_ids.astype(jnp.int32)
    part = _gather(emb_table.T, idx)
    logits_t = _matmul(lm_head_w.T, part)
    return logits_t.T


# speedup vs baseline: 1.0114x; 1.0114x over previous
import functools

import jax
import jax.numpy as jnp
from jax import lax
from jax.experimental import pallas as pl
from jax.experimental.pallas import tpu as pltpu
from jax.experimental.pallas import tpu_sc as plsc

VOCAB = 100000
EMBED = 64
BATCH = 1024

_NC = 2
_NS = 16

_N_WIN = 8
_WIN = 12544
_LAST_OFF = 87552
_N_GROUPS = BATCH // 16
_N_LAYER = _N_WIN // 2

_TILE_V = 4096
_GRID_V = (VOCAB + _TILE_V - 1) // _TILE_V


def _make_gather():
    mesh = plsc.VectorSubcoreMesh(core_axis_name="c", subcore_axis_name="s")

    @functools.partial(
        pl.kernel,
        mesh=mesh,
        out_type=jax.ShapeDtypeStruct((_N_LAYER, EMBED, BATCH), jnp.float32),
        scratch_types=[
            pltpu.VMEM((BATCH,), jnp.int32),
            pltpu.VMEM((8, _WIN), jnp.float32),
            pltpu.VMEM((1, 8, BATCH), jnp.float32),
        ],
        compiler_params=pltpu.CompilerParams(
            use_tc_tiling_on_sc=True,
            disable_bounds_checks=True,
            needs_layout_passes=False,
        ),
    )
    def gather(table_t_hbm, idx_hbm, part_hbm, idx_v, buf_v, out_v):
        cid = lax.axis_index("c")
        sid = lax.axis_index("s")
        row0 = 8 * (4 * cid + sid // 4)
        pltpu.sync_copy(idx_hbm, idx_v)

        for qi in range(2):
            q = 2 * (sid % 4) + qi
            off = jnp.where(q == _N_WIN - 1, _LAST_OFF, q * _WIN)
            own_lo = q * _WIN
            own_hi = jnp.where(
                q == _N_WIN - 2,
                _LAST_OFF,
                jnp.where(q == _N_WIN - 1, VOCAB, (q + 1) * _WIN),
            )
            own_lo = jnp.where(q == _N_WIN - 1, _LAST_OFF, own_lo)
            pltpu.sync_copy(
                table_t_hbm.at[pl.ds(row0, 8), pl.ds(off, _WIN)], buf_v
            )

            def body(g, carry, qi=qi, off=off, own_lo=own_lo, own_hi=own_hi):
                tv = idx_v[pl.ds(g * 16, 16)]
                m = (tv >= own_lo) & (tv < own_hi)
                loc = jnp.clip(tv - off, 0, _WIN - 1)
                for d in range(8):
                    dv = jnp.full((16,), d, jnp.int32)
                    v = plsc.load_gather(buf_v, [dv, loc], mask=m)
                    v = jnp.where(m, v, 0.0)
                    if qi == 0:
                        out_v[0, d, pl.ds(g * 16, 16)] = v
                    else:
                        out_v[0, d, pl.ds(g * 16, 16)] = (
                            out_v[0, d, pl.ds(g * 16, 16)] + v
                        )
                return carry

            lax.fori_loop(0, _N_GROUPS, body, 0)

            if qi == 1:
                pltpu.sync_copy(
                    out_v,
                    part_hbm.at[pl.ds(sid % 4, 1), pl.ds(row0, 8), :],
                )

    return gather


_gather = _make_gather()


def _matmul_body(w_ref, part_ref, out_ref, emb_ref):
    @pl.when(pl.program_id(0) == 0)
    def _():
        emb_ref[...] = jnp.sum(part_ref[...], axis=0)

    out_ref[...] = lax.dot_general(
        w_ref[...],
        emb_ref[...],
        (((0,), (0,)), ((), ())),
        preferred_element_type=jnp.float32,
    )


_matmul = pl.pallas_call(
    _matmul_body,
    grid=(_GRID_V,),
    in_specs=[
        pl.BlockSpec((EMBED, _TILE_V), lambda j: (0, j)),
        pl.BlockSpec((_N_LAYER, EMBED, BATCH), lambda j: (0, 0, 0)),
    ],
    out_specs=pl.BlockSpec((_TILE_V, BATCH), lambda j: (j, 0)),
    out_shape=jax.ShapeDtypeStruct((VOCAB, BATCH), jnp.float32),
    scratch_shapes=[pltpu.VMEM((EMBED, BATCH), jnp.float32)],
    compiler_params=pltpu.CompilerParams(
        dimension_semantics=("arbitrary",),
    ),
)


def kernel(token_ids, emb_table, lm_head_w):
    idx = token_ids.astype(jnp.int32)
    part = _gather(emb_table.T, idx)
    logits_t = _matmul(lm_head_w.T, part)
    return logits_t.T
